# baseline (speedup 1.0000x reference)
import functools

import jax
import jax.numpy as jnp
from jax import lax
from jax.experimental import pallas as pl
from jax.experimental.pallas import tpu as pltpu
from jax.experimental.pallas import tpu_sc as plsc

_EMBED = 64
_SEQ = 200
_NC = 2
_NS = 16
_NW = _NC * _NS
_CHUNK = 128
_NBUF = 4
_LANE = 16
_BBLK = 64


def _gather_body(rows_per_worker, x_ref, tab_ref, out_ref,
                 idx_v, gbuf_v, abuf_v, sem_io, gsem, ssem):
    wid = lax.axis_index("s") * _NC + lax.axis_index("c")
    wbase = pl.multiple_of(wid * rows_per_worker, _CHUNK)
    num_chunks = rows_per_worker // _CHUNK

    pltpu.async_copy(x_ref.at[pl.ds(wbase, rows_per_worker)], idx_v, sem_io).wait()

    def gather(j, b):
        off = pl.multiple_of(j * _CHUNK, _CHUNK)
        idx = idx_v.at[pl.ds(off, _CHUNK)]
        return pltpu.make_async_copy(tab_ref.at[idx], gbuf_v.at[b], gsem.at[b])

    def scatter(j, b):
        off = pl.multiple_of((wbase + j * _CHUNK) // 2, _CHUNK // 2)
        dst = out_ref.at[pl.ds(off, _CHUNK // 2)]
        return pltpu.make_async_copy(abuf_v.at[b], dst, ssem.at[b])

    def repack(b):
        @plsc.parallel_loop(0, _CHUNK, step=2)
        def _(r):
            rr = r // 2
            for u in range(2):
                for k in range(_EMBED // _LANE):
                    abuf_v[b, rr, pl.ds(u * _EMBED + k * _LANE, _LANE)] = (
                        gbuf_v[b, r + u, pl.ds(k * _LANE, _LANE)])

    def do_chunk(j, b, wait_prev, issue_next):
        gather(j, b).wait()
        if wait_prev:
            scatter(j - _NBUF, b).wait()
        repack(b)
        scatter(j, b).start()
        if issue_next:
            gather(j + _NBUF, b).start()

    for j in range(_NBUF):
        gather(j, j).start()
    for j in range(_NBUF):
        do_chunk(j, j, wait_prev=False, issue_next=True)

    def group_body(g, carry):
        for b in range(_NBUF):
            do_chunk(g * _NBUF + b, b, wait_prev=True, issue_next=True)
        return carry

    lax.fori_loop(1, num_chunks // _NBUF - 1, group_body, 0)

    for b in range(_NBUF):
        do_chunk(num_chunks - _NBUF + b, b, wait_prev=True, issue_next=False)
    for b in range(_NBUF):
        scatter(num_chunks - _NBUF + b, b).wait()


def _epilogue_body(g_ref, pos_ref, out_ref):
    a = g_ref[...]
    lo = a[:, :_EMBED]
    hi = a[:, _EMBED:]
    pairs = jnp.stack([lo, hi], axis=1)
    blk = pairs.reshape(_BBLK, _SEQ, _EMBED)
    out_ref[...] = blk + pos_ref[...][None, :, :]


@jax.jit
def _run(x_flat, pos2d, token_table):
    rows = x_flat.shape[0]
    rows_per_worker = rows // _NW
    nseq = rows // _SEQ
    mesh = plsc.VectorSubcoreMesh(core_axis_name="c", subcore_axis_name="s")
    body = functools.partial(_gather_body, rows_per_worker)
    gathered = pl.kernel(
        body,
        mesh=mesh,
        out_type=jax.ShapeDtypeStruct((rows // 2, 2 * _EMBED), jnp.float32),
        scratch_types=[
            pltpu.VMEM((rows_per_worker,), jnp.int32),
            pltpu.VMEM((_NBUF, _CHUNK, _EMBED), jnp.float32),
            pltpu.VMEM((_NBUF, _CHUNK // 2, 2 * _EMBED), jnp.float32),
            pltpu.SemaphoreType.DMA,
            pltpu.SemaphoreType.DMA((_NBUF,)),
            pltpu.SemaphoreType.DMA((_NBUF,)),
        ],
        compiler_params=pltpu.CompilerParams(use_tc_tiling_on_sc=False),
    )(x_flat, token_table)

    lines_per_blk = _BBLK * _SEQ // 2
    out = pl.pallas_call(
        _epilogue_body,
        grid=(nseq // _BBLK,),
        in_specs=[
            pl.BlockSpec((lines_per_blk, 2 * _EMBED), lambda i: (i, 0)),
            pl.BlockSpec((_SEQ, _EMBED), lambda i: (0, 0)),
        ],
        out_specs=pl.BlockSpec((_BBLK, _SEQ, _EMBED), lambda i: (i, 0, 0)),
        out_shape=jax.ShapeDtypeStruct((nseq, _SEQ, _EMBED), jnp.float32),
    )(gathered, pos2d)
    return out


def kernel(x, token_table, pos_table):
    b, l = x.shape
    x_flat = x.reshape(b * l).astype(jnp.int32)
    pos2d = pos_table[:l]
    return _run(x_flat, pos2d, token_table)


# speedup vs baseline: 1.3313x; 1.3313x over previous
import functools

import jax
import jax.numpy as jnp
from jax import lax
from jax.experimental import pallas as pl
from jax.experimental.pallas import tpu as pltpu
from jax.experimental.pallas import tpu_sc as plsc

_EMBED = 64
_SEQ = 200
_NC = 2
_NS = 16
_NW = _NC * _NS
_CHUNK = 128
_NBUF = 3
_LANE = 16
_BBLK = 64
_BLKROWS = _BBLK * _SEQ
_HALF = _BLKROWS // 2
_PAIRS_PER_HALFBLK = _HALF // _CHUNK
_NPAIRS = 100


def _gather_body(rows_per_worker, x_ref, tab_ref, out_ref,
                 idx_v, gbuf_v, abuf_v, sem_io, gsem, ssem):
    wid = lax.axis_index("s") * _NC + lax.axis_index("c")
    wbase = pl.multiple_of(wid * rows_per_worker, _CHUNK)

    pltpu.async_copy(x_ref.at[pl.ds(wbase, rows_per_worker)], idx_v, sem_io).wait()

    def pair_locs(p):
        blk = p // _PAIRS_PER_HALFBLK
        q = p - blk * _PAIRS_PER_HALFBLK
        loc_a = blk * _BLKROWS + q * _CHUNK
        line0 = wid * _BLKROWS + blk * _HALF + q * _CHUNK
        return loc_a, line0

    def gathers(p, s):
        loc_a, _ = pair_locs(p)
        ia = idx_v.at[pl.ds(pl.multiple_of(loc_a, _CHUNK), _CHUNK)]
        ib = idx_v.at[pl.ds(pl.multiple_of(loc_a + _HALF, _CHUNK), _CHUNK)]
        ca = pltpu.make_async_copy(tab_ref.at[ia], gbuf_v.at[s, 0], gsem.at[s])
        cb = pltpu.make_async_copy(tab_ref.at[ib], gbuf_v.at[s, 1], gsem.at[s])
        return ca, cb

    def issue_gathers(p, s):
        ca, cb = gathers(p, s)
        ca.start()
        cb.start()

    def wait_gathers(p, s):
        ca, cb = gathers(p, s)
        ca.wait()
        cb.wait()

    def scatter(p, s):
        _, line0 = pair_locs(p)
        dst = out_ref.at[pl.ds(pl.multiple_of(line0, _CHUNK), _CHUNK)]
        return pltpu.make_async_copy(abuf_v.at[s], dst, ssem.at[s])

    def repack(s):
        @plsc.parallel_loop(0, _CHUNK, step=2)
        def _(t):
            for u in range(2):
                for k in range(_EMBED // _LANE):
                    sl = pl.ds(k * _LANE, _LANE)
                    abuf_v[s, t + u, pl.ds(k * _LANE, _LANE)] = \
                        gbuf_v[s, 0, t + u, sl]
                    abuf_v[s, t + u, pl.ds(_EMBED + k * _LANE, _LANE)] = \
                        gbuf_v[s, 1, t + u, sl]

    def do_pair(p, s, wait_prev, issue_next):
        wait_gathers(p, s)
        if wait_prev:
            scatter(p - _NBUF, s).wait()
        repack(s)
        scatter(p, s).start()
        if issue_next:
            issue_gathers(p + 2, (s + 2) % _NBUF)

    issue_gathers(0, 0)
    issue_gathers(1, 1)
    do_pair(0, 0, wait_prev=False, issue_next=True)
    do_pair(1, 1, wait_prev=False, issue_next=True)
    do_pair(2, 2, wait_prev=False, issue_next=True)

    def group_body(g, carry):
        for b in range(_NBUF):
            do_pair(g * _NBUF + b, b, wait_prev=True, issue_next=True)
        return carry

    lax.fori_loop(1, (_NPAIRS - 4) // _NBUF, group_body, 0)

    do_pair(_NPAIRS - 4, 0, wait_prev=True, issue_next=True)
    do_pair(_NPAIRS - 3, 1, wait_prev=True, issue_next=True)
    do_pair(_NPAIRS - 2, 2, wait_prev=True, issue_next=False)
    do_pair(_NPAIRS - 1, 0, wait_prev=True, issue_next=False)
    scatter(_NPAIRS - 3, 1).wait()
    scatter(_NPAIRS - 2, 2).wait()
    scatter(_NPAIRS - 1, 0).wait()


def _epilogue_body(g_ref, pos_ref, out_ref):
    a = g_ref[...]
    lo = a[:, :_EMBED]
    hi = a[:, _EMBED:]
    half = _BBLK // 2
    blk = jnp.concatenate(
        [lo.reshape(half, _SEQ, _EMBED), hi.reshape(half, _SEQ, _EMBED)],
        axis=0)
    out_ref[...] = blk + pos_ref[...][None, :, :]


@jax.jit
def _run(x_flat, pos2d, token_table):
    rows = x_flat.shape[0]
    rows_per_worker = rows // _NW
    nseq = rows // _SEQ
    mesh = plsc.VectorSubcoreMesh(core_axis_name="c", subcore_axis_name="s")
    body = functools.partial(_gather_body, rows_per_worker)
    gathered = pl.kernel(
        body,
        mesh=mesh,
        out_type=jax.ShapeDtypeStruct((rows // 2, 2 * _EMBED), jnp.float32),
        scratch_types=[
            pltpu.VMEM((rows_per_worker,), jnp.int32),
            pltpu.VMEM((_NBUF, 2, _CHUNK, _EMBED), jnp.float32),
            pltpu.VMEM((_NBUF, _CHUNK, 2 * _EMBED), jnp.float32),
            pltpu.SemaphoreType.DMA,
            pltpu.SemaphoreType.DMA((_NBUF,)),
            pltpu.SemaphoreType.DMA((_NBUF,)),
        ],
        compiler_params=pltpu.CompilerParams(use_tc_tiling_on_sc=False),
    )(x_flat, token_table)

    out = pl.pallas_call(
        _epilogue_body,
        grid=(nseq // _BBLK,),
        in_specs=[
            pl.BlockSpec((_HALF, 2 * _EMBED), lambda i: (i, 0)),
            pl.BlockSpec((_SEQ, _EMBED), lambda i: (0, 0)),
        ],
        out_specs=pl.BlockSpec((_BBLK, _SEQ, _EMBED), lambda i: (i, 0, 0)),
        out_shape=jax.ShapeDtypeStruct((nseq, _SEQ, _EMBED), jnp.float32),
    )(gathered, pos2d)
    return out


def kernel(x, token_table, pos_table):
    b, l = x.shape
    x_flat = x.reshape(b * l).astype(jnp.int32)
    pos2d = pos_table[:l]
    return _run(x_flat, pos2d, token_table)
